# baseline (speedup 1.0000x reference)
import functools

import jax
import jax.numpy as jnp
from jax.experimental import pallas as pl
from jax.experimental.pallas import tpu as pltpu

N = 4096
K = 4
D = 256
H = 256
BN = 128
NB = N // BN

_SELU_ALPHA = 1.6732632423543772
_SELU_SCALE = 1.0507009873554805


def _selu(v):
    return _SELU_SCALE * jnp.where(v > 0, v, _SELU_ALPHA * (jnp.exp(v) - 1.0))


def _mm_t(a, w):
    return jax.lax.dot_general(a, w, (((1,), (1,)), ((), ())),
                               preferred_element_type=jnp.float32)


def _split(v):
    vh = v.astype(jnp.bfloat16)
    vl = (v - vh.astype(jnp.float32)).astype(jnp.bfloat16)
    return vh, vl


def _layer_body(ah_ref, al_ref, xh_ref, xl_ref, wih_ref, whh_ref,
                bih_ref, bhh_ref, g_ref, b_ref, out_ref):
    f32 = jnp.float32
    ah = ah_ref[...].reshape(K * BN, N)
    al = al_ref[...].reshape(K * BN, N)
    xh = xh_ref[...]
    xl = xl_ref[...]
    hx = (jnp.dot(ah, xh, preferred_element_type=f32)
          + jnp.dot(ah, xl, preferred_element_type=f32)
          + jnp.dot(al, xh, preferred_element_type=f32))
    hx = _selu(hx)
    gi = _mm_t(hx, wih_ref[...]) + bih_ref[...]

    bhh = bhh_ref[...]
    h = jnp.zeros((BN, H), dtype=jnp.float32)
    s = jnp.zeros((BN, H), dtype=jnp.float32)
    for t in range(K):
        git = gi[t * BN:(t + 1) * BN]
        if t == 0:
            gh = jnp.broadcast_to(bhh, (BN, 3 * H))
        else:
            gh = _mm_t(h, whh_ref[...]) + bhh
        r = jax.nn.sigmoid(git[:, 0:H] + gh[:, 0:H])
        z = jax.nn.sigmoid(git[:, H:2 * H] + gh[:, H:2 * H])
        n = jnp.tanh(git[:, 2 * H:] + r * gh[:, 2 * H:])
        h = (1.0 - z) * n + z * h
        s = s + h

    mu = jnp.mean(s, axis=-1, keepdims=True)
    var = jnp.mean((s - mu) ** 2, axis=-1, keepdims=True)
    out_ref[...] = (s - mu) * jax.lax.rsqrt(var + 1e-5) * g_ref[...] + b_ref[...]


@functools.partial(jax.jit, static_argnames=())
def _diffusion_layer(x, ah, al, wih, whh, bih, bhh, g, b):
    xh, xl = _split(x)
    return pl.pallas_call(
        _layer_body,
        grid=(NB,),
        in_specs=[
            pl.BlockSpec((K, BN, N), lambda i: (0, i, 0)),
            pl.BlockSpec((K, BN, N), lambda i: (0, i, 0)),
            pl.BlockSpec((N, D), lambda i: (0, 0)),
            pl.BlockSpec((N, D), lambda i: (0, 0)),
            pl.BlockSpec((3 * H, D), lambda i: (0, 0)),
            pl.BlockSpec((3 * H, H), lambda i: (0, 0)),
            pl.BlockSpec((1, 3 * H), lambda i: (0, 0)),
            pl.BlockSpec((1, 3 * H), lambda i: (0, 0)),
            pl.BlockSpec((1, H), lambda i: (0, 0)),
            pl.BlockSpec((1, H), lambda i: (0, 0)),
        ],
        out_specs=pl.BlockSpec((BN, H), lambda i: (i, 0)),
        out_shape=jax.ShapeDtypeStruct((N, H), jnp.float32),
        compiler_params=pltpu.CompilerParams(
            dimension_semantics=("parallel",),
        ),
    )(ah, al, xh, xl, wih, whh, bih, bhh, g, b)


def kernel(x, adj_list, W_ih0, W_hh0, b_ih0, b_hh0, ln_g0, ln_b0,
           W_ih1, W_hh1, b_ih1, b_hh1, ln_g1, ln_b1):
    ah, al = _split(adj_list)
    h = _diffusion_layer(x, ah, al, W_ih0, W_hh0,
                         b_ih0.reshape(1, -1), b_hh0.reshape(1, -1),
                         ln_g0.reshape(1, -1), ln_b0.reshape(1, -1))
    h = _diffusion_layer(h, ah, al, W_ih1, W_hh1,
                         b_ih1.reshape(1, -1), b_hh1.reshape(1, -1),
                         ln_g1.reshape(1, -1), ln_b1.reshape(1, -1))
    return h


# speedup vs baseline: 1.4693x; 1.4693x over previous
import functools

import jax
import jax.numpy as jnp
from jax.experimental import pallas as pl
from jax.experimental.pallas import tpu as pltpu

N = 4096
K = 4
D = 256
H = 256
BN = 128
NB = N // BN

_SELU_ALPHA = 1.6732632423543772
_SELU_SCALE = 1.0507009873554805


def _selu(v):
    return _SELU_SCALE * jnp.where(v > 0, v, _SELU_ALPHA * (jnp.exp(v) - 1.0))


def _mm_t(a, w):
    return jax.lax.dot_general(a, w, (((1,), (1,)), ((), ())),
                               preferred_element_type=jnp.float32)


def _split(v):
    vh = v.astype(jnp.bfloat16)
    vl = (v - vh.astype(jnp.float32)).astype(jnp.bfloat16)
    return vh, vl


def _split_in_kernel(v):
    vh = v.astype(jnp.bfloat16)
    vl = (v - vh.astype(jnp.float32)).astype(jnp.bfloat16)
    return vh, vl


def _layer_body(adj_ref, xh_ref, xl_ref, wih_ref, whh_ref,
                bih_ref, bhh_ref, g_ref, b_ref, out_ref):
    f32 = jnp.float32
    a = adj_ref[...].reshape(K * BN, N)
    ah, al = _split_in_kernel(a)
    xh = xh_ref[...]
    xl = xl_ref[...]
    hx = (jnp.dot(ah, xh, preferred_element_type=f32)
          + jnp.dot(ah, xl, preferred_element_type=f32)
          + jnp.dot(al, xh, preferred_element_type=f32))
    hx = _selu(hx)
    gi = _mm_t(hx, wih_ref[...]) + bih_ref[...]

    bhh = bhh_ref[...]
    h = jnp.zeros((BN, H), dtype=jnp.float32)
    s = jnp.zeros((BN, H), dtype=jnp.float32)
    for t in range(K):
        git = gi[t * BN:(t + 1) * BN]
        if t == 0:
            gh = jnp.broadcast_to(bhh, (BN, 3 * H))
        else:
            gh = _mm_t(h, whh_ref[...]) + bhh
        r = jax.nn.sigmoid(git[:, 0:H] + gh[:, 0:H])
        z = jax.nn.sigmoid(git[:, H:2 * H] + gh[:, H:2 * H])
        n = jnp.tanh(git[:, 2 * H:] + r * gh[:, 2 * H:])
        h = (1.0 - z) * n + z * h
        s = s + h

    mu = jnp.mean(s, axis=-1, keepdims=True)
    var = jnp.mean((s - mu) ** 2, axis=-1, keepdims=True)
    out_ref[...] = (s - mu) * jax.lax.rsqrt(var + 1e-5) * g_ref[...] + b_ref[...]


@functools.partial(jax.jit, static_argnames=())
def _diffusion_layer(x, adj_list, wih, whh, bih, bhh, g, b):
    xh, xl = _split(x)
    return pl.pallas_call(
        _layer_body,
        grid=(NB,),
        in_specs=[
            pl.BlockSpec((K, BN, N), lambda i: (0, i, 0)),
            pl.BlockSpec((N, D), lambda i: (0, 0)),
            pl.BlockSpec((N, D), lambda i: (0, 0)),
            pl.BlockSpec((3 * H, D), lambda i: (0, 0)),
            pl.BlockSpec((3 * H, H), lambda i: (0, 0)),
            pl.BlockSpec((1, 3 * H), lambda i: (0, 0)),
            pl.BlockSpec((1, 3 * H), lambda i: (0, 0)),
            pl.BlockSpec((1, H), lambda i: (0, 0)),
            pl.BlockSpec((1, H), lambda i: (0, 0)),
        ],
        out_specs=pl.BlockSpec((BN, H), lambda i: (i, 0)),
        out_shape=jax.ShapeDtypeStruct((N, H), jnp.float32),
        compiler_params=pltpu.CompilerParams(
            dimension_semantics=("parallel",),
        ),
    )(adj_list, xh, xl, wih, whh, bih, bhh, g, b)


def kernel(x, adj_list, W_ih0, W_hh0, b_ih0, b_hh0, ln_g0, ln_b0,
           W_ih1, W_hh1, b_ih1, b_hh1, ln_g1, ln_b1):
    h = _diffusion_layer(x, adj_list, W_ih0, W_hh0,
                         b_ih0.reshape(1, -1), b_hh0.reshape(1, -1),
                         ln_g0.reshape(1, -1), ln_b0.reshape(1, -1))
    h = _diffusion_layer(h, adj_list, W_ih1, W_hh1,
                         b_ih1.reshape(1, -1), b_hh1.reshape(1, -1),
                         ln_g1.reshape(1, -1), ln_b1.reshape(1, -1))
    return h


# speedup vs baseline: 2.7321x; 1.8595x over previous
import functools

import jax
import jax.numpy as jnp
from jax.experimental import pallas as pl
from jax.experimental.pallas import tpu as pltpu

N = 4096
K = 4
D = 256
H = 256
BN = 256
NB = N // BN

_SELU_ALPHA = 1.6732632423543772
_SELU_SCALE = 1.0507009873554805


def _selu(v):
    return _SELU_SCALE * jnp.where(v > 0, v, _SELU_ALPHA * (jnp.exp(v) - 1.0))


def _mm_t(a, w):
    return jax.lax.dot_general(a, w, (((1,), (1,)), ((), ())),
                               preferred_element_type=jnp.float32)


def _layer_body(adj_ref, x_ref, wih_ref, whh_ref,
                bih_ref, bhh_ref, g_ref, b_ref, out_ref):
    f32 = jnp.float32
    a = adj_ref[...].reshape(K * BN, N)
    hx = jnp.dot(a, x_ref[...], preferred_element_type=f32)
    hx = _selu(hx)
    gi = _mm_t(hx, wih_ref[...]) + bih_ref[...]

    bhh = bhh_ref[...]
    h = jnp.zeros((BN, H), dtype=jnp.float32)
    s = jnp.zeros((BN, H), dtype=jnp.float32)
    for t in range(K):
        git = gi[t * BN:(t + 1) * BN]
        if t == 0:
            gh = jnp.broadcast_to(bhh, (BN, 3 * H))
        else:
            gh = _mm_t(h, whh_ref[...]) + bhh
        r = jax.nn.sigmoid(git[:, 0:H] + gh[:, 0:H])
        z = jax.nn.sigmoid(git[:, H:2 * H] + gh[:, H:2 * H])
        n = jnp.tanh(git[:, 2 * H:] + r * gh[:, 2 * H:])
        h = (1.0 - z) * n + z * h
        s = s + h

    mu = jnp.mean(s, axis=-1, keepdims=True)
    var = jnp.mean((s - mu) ** 2, axis=-1, keepdims=True)
    out_ref[...] = (s - mu) * jax.lax.rsqrt(var + 1e-5) * g_ref[...] + b_ref[...]


@functools.partial(jax.jit, static_argnames=())
def _diffusion_layer(x, adj_list, wih, whh, bih, bhh, g, b):
    return pl.pallas_call(
        _layer_body,
        grid=(NB,),
        in_specs=[
            pl.BlockSpec((K, BN, N), lambda i: (0, i, 0)),
            pl.BlockSpec((N, D), lambda i: (0, 0)),
            pl.BlockSpec((3 * H, D), lambda i: (0, 0)),
            pl.BlockSpec((3 * H, H), lambda i: (0, 0)),
            pl.BlockSpec((1, 3 * H), lambda i: (0, 0)),
            pl.BlockSpec((1, 3 * H), lambda i: (0, 0)),
            pl.BlockSpec((1, H), lambda i: (0, 0)),
            pl.BlockSpec((1, H), lambda i: (0, 0)),
        ],
        out_specs=pl.BlockSpec((BN, H), lambda i: (i, 0)),
        out_shape=jax.ShapeDtypeStruct((N, H), jnp.float32),
        compiler_params=pltpu.CompilerParams(
            dimension_semantics=("parallel",),
        ),
    )(adj_list, x, wih, whh, bih, bhh, g, b)


def kernel(x, adj_list, W_ih0, W_hh0, b_ih0, b_hh0, ln_g0, ln_b0,
           W_ih1, W_hh1, b_ih1, b_hh1, ln_g1, ln_b1):
    h = _diffusion_layer(x, adj_list, W_ih0, W_hh0,
                         b_ih0.reshape(1, -1), b_hh0.reshape(1, -1),
                         ln_g0.reshape(1, -1), ln_b0.reshape(1, -1))
    h = _diffusion_layer(h, adj_list, W_ih1, W_hh1,
                         b_ih1.reshape(1, -1), b_hh1.reshape(1, -1),
                         ln_g1.reshape(1, -1), ln_b1.reshape(1, -1))
    return h
